# baseline (speedup 1.0000x reference)
import jax
import jax.numpy as jnp
from jax.experimental import pallas as pl
from jax.experimental.pallas import tpu as pltpu

_LABELNUM = 1000
_BLK = 2048


def _onehot_block(idx_ref, out_ref):
    idx = idx_ref[...]
    cols = jax.lax.broadcasted_iota(jnp.int32, out_ref.shape, 1)
    out_ref[...] = (cols == idx).astype(jnp.float32)


def kernel(input):
    B = input.shape[0]
    idx = input.astype(jnp.int32)
    return pl.pallas_call(
        _onehot_block,
        grid=(B // _BLK,),
        in_specs=[pl.BlockSpec((_BLK, 1), lambda i: (i, 0))],
        out_specs=pl.BlockSpec((_BLK, _LABELNUM), lambda i: (i, 0)),
        out_shape=jax.ShapeDtypeStruct((B, _LABELNUM), jnp.float32),
        compiler_params=pltpu.CompilerParams(
            dimension_semantics=("parallel",),
        ),
    )(idx)
